# Initial kernel scaffold; baseline (speedup 1.0000x reference)
import jax
import jax.numpy as jnp
from jax.experimental import pallas as pl


def kernel(node_feat, edge_index1, edge_index2, edge_weight1, edge_weight2, W_self1, W_neigh1, b1, W_self2, W_neigh2, b2):
    raise NotImplementedError("write your pallas kernel here")


# speedup vs baseline: 7.8878x; 7.8878x over previous
import dataclasses
import functools

import jax
import jax.numpy as jnp
from jax import lax
from jax.experimental import pallas as pl
from jax.experimental.pallas import tpu as pltpu
from jax.experimental.pallas import tpu_sc as plsc

_N_SRC = 10000
_N_DST1 = 5000
_N_DST2 = 1000
_E1 = 320000
_E2 = 160000
_D_IN = 128
_HID = 16
_D_OUT = 128

_NC = 2
_NS = 16
_NW = _NC * _NS
_GRP = 128

_E1P = 327680
_E2P = 163840
_G1 = _E1P // (_NW * _GRP)
_G2 = _E2P // (_NW * _GRP)
_ND1P = 5120
_ND2P = 1024


def _leaky(x):
    return jnp.where(x > 0, x, 0.01 * x)


def _proj_body(h_ref, wn_ref, ws_ref, p_ref, s_ref):
    h = h_ref[...]
    p_ref[...] = jnp.dot(h, wn_ref[...], preferred_element_type=jnp.float32)
    s_ref[...] = jnp.dot(h[:_N_DST1], ws_ref[...],
                         preferred_element_type=jnp.float32)


def _h1_body(s_ref, agg_ref, deg_ref, b_ref, h1_ref):
    a = agg_ref[0, :_N_DST1, :] + agg_ref[1, :_N_DST1, :]
    d = deg_ref[0, :_N_DST1, :] + deg_ref[1, :_N_DST1, :]
    x = s_ref[...] + a / jnp.maximum(d, 1.0) + b_ref[...]
    h1_ref[...] = _leaky(x)


def _out_body(h1_ref, agg_ref, deg_ref, ws_ref, wn_ref, b_ref, o_ref):
    a = agg_ref[0, :_N_DST2, :] + agg_ref[1, :_N_DST2, :]
    d = deg_ref[0, :_N_DST2, :] + deg_ref[1, :_N_DST2, :]
    hn = a / jnp.maximum(d, 1.0)
    x = (jnp.dot(h1_ref[:_N_DST2], ws_ref[...],
                 preferred_element_type=jnp.float32)
         + jnp.dot(hn, wn_ref[...], preferred_element_type=jnp.float32)
         + b_ref[...])
    o_ref[...] = _leaky(x)


def _make_sc_aggregate(n_dst_pad, groups_per_tile):
    gpt = groups_per_tile
    zr = n_dst_pad // _NS

    mesh = plsc.VectorSubcoreMesh(core_axis_name="core",
                                  subcore_axis_name="subcore")

    def body(p_hbm, src_hbm, dst_hbm, ew_hbm, agg_out, deg_out,
             src_v, dst_v, ew_v, rows_v, ones_v, zrow_v, agg_sp, deg_sp):
        cid = lax.axis_index("core")
        sid = lax.axis_index("subcore")
        wid = sid * _NC + cid

        @pl.loop(0, zr)
        def _(i):
            zrow_v[i, :] = jnp.zeros((16,), jnp.float32)

        @pl.loop(0, _GRP)
        def _(i):
            ones_v[i, :] = jnp.ones((16,), jnp.float32)

        pltpu.sync_copy(zrow_v, agg_sp.at[pl.ds(sid * zr, zr)])
        pltpu.sync_copy(zrow_v, deg_sp.at[pl.ds(sid * zr, zr)])
        plsc.subcore_barrier()

        pltpu.sync_copy(src_hbm.at[pl.ds(wid * gpt, gpt)], src_v)
        pltpu.sync_copy(dst_hbm.at[pl.ds(wid * gpt, gpt)], dst_v)
        pltpu.sync_copy(ew_hbm.at[pl.ds(wid * gpt * _GRP, gpt * _GRP)], ew_v)

        lanes = lax.iota(jnp.int32, 16)

        @pl.loop(0, gpt)
        def _(g):
            pltpu.sync_copy(p_hbm.at[src_v.at[g]], rows_v)

            @pl.loop(0, _GRP // 16)
            def _(j):
                w = ew_v[pl.ds(g * _GRP + j * 16, 16)]
                eidx = j * 16 + lanes
                for f in range(16):
                    fidx = jnp.full((16,), f, jnp.int32)
                    col = plsc.load_gather(rows_v, [eidx, fidx])
                    plsc.store_scatter(rows_v, [eidx, fidx], col * w)

            pltpu.sync_copy(rows_v, agg_sp.at[dst_v.at[g]], add=True)
            pltpu.sync_copy(ones_v, deg_sp.at[dst_v.at[g]], add=True)

        plsc.subcore_barrier()

        pltpu.sync_copy(agg_sp.at[pl.ds(sid * zr, zr)],
                        agg_out.at[cid, pl.ds(sid * zr, zr)])
        pltpu.sync_copy(deg_sp.at[pl.ds(sid * zr, zr)],
                        deg_out.at[cid, pl.ds(sid * zr, zr)])

    out_type = [jax.ShapeDtypeStruct((_NC, n_dst_pad, 16), jnp.float32),
                jax.ShapeDtypeStruct((_NC, n_dst_pad, 16), jnp.float32)]
    scratch = [
        pltpu.VMEM((gpt, _GRP), jnp.int32),
        pltpu.VMEM((gpt, _GRP), jnp.int32),
        pltpu.VMEM((gpt * _GRP,), jnp.float32),
        pltpu.VMEM((_GRP, 16), jnp.float32),
        pltpu.VMEM((_GRP, 16), jnp.float32),
        pltpu.VMEM((zr, 16), jnp.float32),
        pltpu.VMEM_SHARED((n_dst_pad, 16), jnp.float32),
        pltpu.VMEM_SHARED((n_dst_pad, 16), jnp.float32),
    ]
    cp = pltpu.CompilerParams(needs_layout_passes=False,
                              use_tc_tiling_on_sc=False)
    return pl.kernel(body, out_type=out_type, mesh=mesh,
                     scratch_types=scratch, compiler_params=cp)


_sc_agg1 = _make_sc_aggregate(_ND1P, _G1)
_sc_agg2 = _make_sc_aggregate(_ND2P, _G2)


def _pad_edges(src, dst, ew, e_pad, n_dst):
    e = src.shape[0]
    src = jnp.pad(src, (0, e_pad - e)).reshape(-1, _GRP)
    dst = jnp.pad(dst, (0, e_pad - e),
                  constant_values=n_dst).reshape(-1, _GRP)
    ew = jnp.pad(ew, (0, e_pad - e))
    return src, dst, ew


def kernel(node_feat, edge_index1, edge_index2, edge_weight1, edge_weight2,
           W_self1, W_neigh1, b1, W_self2, W_neigh2, b2):
    h = node_feat.reshape(_N_SRC, _D_IN)

    src1, dst1, ew1 = _pad_edges(edge_index1[0], edge_index1[1],
                                 edge_weight1, _E1P, _N_DST1)
    src2, dst2, ew2 = _pad_edges(edge_index2[0], edge_index2[1],
                                 edge_weight2, _E2P, _N_DST2)

    p1, s1 = pl.pallas_call(
        _proj_body,
        out_shape=[jax.ShapeDtypeStruct((_N_SRC, _HID), jnp.float32),
                   jax.ShapeDtypeStruct((_N_DST1, _HID), jnp.float32)],
    )(h, W_neigh1, W_self1)

    agg1, deg1 = _sc_agg1(p1, src1, dst1, ew1)

    h1 = pl.pallas_call(
        _h1_body,
        out_shape=jax.ShapeDtypeStruct((_N_DST1, _HID), jnp.float32),
    )(s1, agg1, deg1, b1.reshape(1, _HID))

    agg2, deg2 = _sc_agg2(h1, src2, dst2, ew2)

    out = pl.pallas_call(
        _out_body,
        out_shape=jax.ShapeDtypeStruct((_N_DST2, _D_OUT), jnp.float32),
    )(h1, agg2, deg2, W_self2, W_neigh2, b2.reshape(1, _D_OUT))

    return out.reshape(1, _N_DST2, _D_OUT)
